# Initial kernel scaffold; baseline (speedup 1.0000x reference)
import jax
import jax.numpy as jnp
from jax.experimental import pallas as pl


def kernel(x, edge_index, W1, b1, gamma1, beta1, a1, W2, b2, gamma2, beta2, a2):
    raise NotImplementedError("write your pallas kernel here")


# speedup vs baseline: 8.4010x; 8.4010x over previous
import functools

import jax
import jax.numpy as jnp
from jax import lax
from jax.experimental import pallas as pl
from jax.experimental.pallas import tpu as pltpu
from jax.experimental.pallas import tpu_sc as plsc

EPS = 1e-5
LANES = 16
CHUNK = 128


def _sc_mesh():
    return plsc.VectorSubcoreMesh(core_axis_name="c", subcore_axis_name="s")


@functools.lru_cache(maxsize=None)
def _deg_kernel(NROWS, D, NC, NS, NCHUNK):
    rpt = NROWS // NS

    @functools.partial(
        pl.kernel,
        mesh=_sc_mesh(),
        out_type=jax.ShapeDtypeStruct((NC, NROWS, D), jnp.float32),
        scratch_types=[
            pltpu.VMEM((NCHUNK, CHUNK), jnp.int32),
            pltpu.VMEM((CHUNK, D), jnp.float32),
            pltpu.VMEM_SHARED((NROWS, D), jnp.float32),
        ],
    )
    def deg(dst_hbm, ones_hbm, zeros_hbm, out_hbm, dstblk, ones_v, acc):
        c = lax.axis_index("c")
        s = lax.axis_index("s")
        w = c * NS + s
        pltpu.sync_copy(dst_hbm.at[w], dstblk)
        pltpu.sync_copy(ones_hbm, ones_v)
        pltpu.sync_copy(zeros_hbm.at[pl.ds(s * rpt, rpt)],
                        acc.at[pl.ds(s * rpt, rpt)])
        plsc.subcore_barrier()

        def step(j, carry):
            pltpu.sync_copy(ones_v, acc.at[dstblk.at[j]], add=True)
            return carry

        lax.fori_loop(0, NCHUNK, step, 0)
        plsc.subcore_barrier()
        pltpu.sync_copy(acc.at[pl.ds(s * rpt, rpt)],
                        out_hbm.at[c, pl.ds(s * rpt, rpt)])

    return deg


@functools.lru_cache(maxsize=None)
def _agg_kernel(NROWS, D, NC, NS, NCHUNK):
    rpt = NROWS // NS

    @functools.partial(
        pl.kernel,
        mesh=_sc_mesh(),
        out_type=jax.ShapeDtypeStruct((NC, NROWS, D), jnp.float32),
        scratch_types=[
            pltpu.VMEM((NCHUNK, CHUNK), jnp.int32),
            pltpu.VMEM((NCHUNK, CHUNK), jnp.int32),
            pltpu.VMEM((CHUNK, D), jnp.float32),
            pltpu.VMEM_SHARED((NROWS, D), jnp.float32),
            pltpu.SemaphoreType.DMA,
        ],
    )
    def agg(table_hbm, src_hbm, dst_hbm, zeros_hbm, out_hbm,
            srcblk, dstblk, rowbuf, acc, sem):
        c = lax.axis_index("c")
        s = lax.axis_index("s")
        w = c * NS + s
        pltpu.sync_copy(src_hbm.at[w], srcblk)
        pltpu.sync_copy(dst_hbm.at[w], dstblk)
        pltpu.sync_copy(zeros_hbm.at[pl.ds(s * rpt, rpt)],
                        acc.at[pl.ds(s * rpt, rpt)])
        plsc.subcore_barrier()

        def step(j, carry):
            pltpu.async_copy(table_hbm.at[srcblk.at[j]], rowbuf, sem).wait()
            pltpu.sync_copy(rowbuf, acc.at[dstblk.at[j]], add=True)
            return carry

        lax.fori_loop(0, NCHUNK, step, 0)
        plsc.subcore_barrier()
        pltpu.sync_copy(acc.at[pl.ds(s * rpt, rpt)],
                        out_hbm.at[c, pl.ds(s * rpt, rpt)])

    return agg


def _pre_body(degp_ref, x_ref, w1_ref, scaled_ref, dis_ref):
    N = x_ref.shape[0]
    cnt = degp_ref[0, 0:N, 0:1] + degp_ref[1, 0:N, 0:1]
    dis = lax.rsqrt(cnt + 1.0)
    dis_ref[...] = dis
    xw = jnp.dot(x_ref[...], w1_ref[...], preferred_element_type=jnp.float32)
    scaled_ref[...] = xw * dis


def _bn_prelu(p_ref, scaled_ref, dis_ref, b_ref, g_ref, be_ref, a_ref):
    dis = dis_ref[...]
    N = scaled_ref.shape[0]
    h = (p_ref[0, 0:N] + p_ref[1, 0:N] + scaled_ref[...]) * dis + b_ref[...]
    mean = jnp.mean(h, axis=0, keepdims=True)
    d = h - mean
    var = jnp.mean(d * d, axis=0, keepdims=True)
    hn = d * lax.rsqrt(var + EPS) * g_ref[...] + be_ref[...]
    aa = a_ref[...]
    return jnp.where(hn >= 0.0, hn, aa * hn), dis


def _mid_body(p_ref, scaled_ref, dis_ref, b_ref, g_ref, be_ref, a_ref,
              w2_ref, out_ref):
    h, dis = _bn_prelu(p_ref, scaled_ref, dis_ref, b_ref, g_ref, be_ref, a_ref)
    out_ref[...] = jnp.dot(h, w2_ref[...],
                           preferred_element_type=jnp.float32) * dis


def _post_body(p_ref, scaled_ref, dis_ref, b_ref, g_ref, be_ref, a_ref,
               out_ref):
    h, _ = _bn_prelu(p_ref, scaled_ref, dis_ref, b_ref, g_ref, be_ref, a_ref)
    out_ref[...] = h


def kernel(x, edge_index, W1, b1, gamma1, beta1, a1, W2, b2, gamma2, beta2, a2):
    N, _ = x.shape
    D = W1.shape[1]
    E = edge_index.shape[1]
    info = plsc.get_sparse_core_info()
    NC, NS = info.num_cores, info.num_subcores
    NW = NC * NS
    NCHUNK = -(-E // (NW * CHUNK))
    NCHUNK += NCHUNK % 2
    EPAD = NW * NCHUNK * CHUNK
    NROWS = -(-(N + 1) // (NS * 8)) * (NS * 8)

    src = edge_index[0].astype(jnp.int32)
    dst = edge_index[1].astype(jnp.int32)
    src3 = jnp.concatenate(
        [src, jnp.zeros((EPAD - E,), jnp.int32)]).reshape(NW, NCHUNK, CHUNK)
    dst3 = jnp.concatenate(
        [dst, jnp.full((EPAD - E,), N, jnp.int32)]).reshape(NW, NCHUNK, CHUNK)
    zeros128 = jnp.zeros((NROWS, D), jnp.float32)
    ones_rows = jnp.ones((CHUNK, D), jnp.float32)

    degp = _deg_kernel(NROWS, D, NC, NS, NCHUNK)(dst3, ones_rows, zeros128)

    scaled1, dis = pl.pallas_call(
        _pre_body,
        out_shape=[jax.ShapeDtypeStruct((N, D), jnp.float32),
                   jax.ShapeDtypeStruct((N, 1), jnp.float32)],
    )(degp, x, W1)

    agg = _agg_kernel(NROWS, D, NC, NS, NCHUNK)
    b1r, g1r, be1r = b1.reshape(1, D), gamma1.reshape(1, D), beta1.reshape(1, D)
    b2r, g2r, be2r = b2.reshape(1, D), gamma2.reshape(1, D), beta2.reshape(1, D)
    a1r, a2r = a1.reshape(1, 1), a2.reshape(1, 1)

    p1 = agg(scaled1, src3, dst3, zeros128)
    scaled2 = pl.pallas_call(
        _mid_body,
        out_shape=jax.ShapeDtypeStruct((N, D), jnp.float32),
    )(p1, scaled1, dis, b1r, g1r, be1r, a1r, W2)

    p2 = agg(scaled2, src3, dst3, zeros128)
    out = pl.pallas_call(
        _post_body,
        out_shape=jax.ShapeDtypeStruct((N, D), jnp.float32),
    )(p2, scaled2, dis, b2r, g2r, be2r, a2r)

    return out


# speedup vs baseline: 9.1530x; 1.0895x over previous
import functools

import jax
import jax.numpy as jnp
from jax import lax
from jax.experimental import pallas as pl
from jax.experimental.pallas import tpu as pltpu
from jax.experimental.pallas import tpu_sc as plsc

EPS = 1e-5
LANES = 16
CHUNK = 128


def _sc_mesh():
    return plsc.VectorSubcoreMesh(core_axis_name="c", subcore_axis_name="s")


@functools.lru_cache(maxsize=None)
def _deg_kernel(NROWS, D, NC, NS, NCHUNK):
    rpt = NROWS // NS

    @functools.partial(
        pl.kernel,
        mesh=_sc_mesh(),
        out_type=jax.ShapeDtypeStruct((NC, NROWS, D), jnp.float32),
        scratch_types=[
            pltpu.VMEM((NCHUNK, CHUNK), jnp.int32),
            pltpu.VMEM((CHUNK, D), jnp.float32),
            pltpu.VMEM_SHARED((NROWS, D), jnp.float32),
        ],
    )
    def deg(dst_hbm, ones_hbm, zeros_hbm, out_hbm, dstblk, ones_v, acc):
        c = lax.axis_index("c")
        s = lax.axis_index("s")
        w = c * NS + s
        pltpu.sync_copy(dst_hbm.at[w], dstblk)
        pltpu.sync_copy(ones_hbm, ones_v)
        pltpu.sync_copy(zeros_hbm.at[pl.ds(s * rpt, rpt)],
                        acc.at[pl.ds(s * rpt, rpt)])
        plsc.subcore_barrier()

        def step(j, carry):
            pltpu.sync_copy(ones_v, acc.at[dstblk.at[j]], add=True)
            return carry

        lax.fori_loop(0, NCHUNK, step, 0)
        plsc.subcore_barrier()
        pltpu.sync_copy(acc.at[pl.ds(s * rpt, rpt)],
                        out_hbm.at[c, pl.ds(s * rpt, rpt)])

    return deg


@functools.lru_cache(maxsize=None)
def _agg_kernel(NROWS, D, NC, NS, NCHUNK):
    rpt = NROWS // NS
    H = NCHUNK // 2

    @functools.partial(
        pl.kernel,
        mesh=_sc_mesh(),
        out_type=jax.ShapeDtypeStruct((NC, NROWS, D), jnp.float32),
        scratch_types=[
            pltpu.VMEM((NCHUNK, CHUNK), jnp.int32),
            pltpu.VMEM((H, CHUNK), jnp.int32),
            pltpu.VMEM((CHUNK, D), jnp.float32),
            pltpu.VMEM((CHUNK, D), jnp.float32),
            pltpu.VMEM_SHARED((NROWS, D), jnp.float32),
            pltpu.SemaphoreType.DMA,
            pltpu.SemaphoreType.DMA,
            pltpu.SemaphoreType.DMA,
            pltpu.SemaphoreType.DMA,
        ],
    )
    def agg(table_hbm, src_hbm, dst_hbm, zeros_hbm, out_hbm,
            srcblk, dstblk, rb0, rb1, acc, gs0, gs1, ss0, ss1):
        rbs = (rb0, rb1)
        gsems = (gs0, gs1)
        ssems = (ss0, ss1)
        c = lax.axis_index("c")
        s = lax.axis_index("s")
        w = c * NS + s
        pltpu.sync_copy(src_hbm.at[w], srcblk)
        pltpu.sync_copy(dst_hbm.at[w, pl.ds(0, H)], dstblk)
        for b in range(2):
            pltpu.async_copy(table_hbm.at[srcblk.at[b]], rbs[b], gsems[b])
        pltpu.sync_copy(zeros_hbm.at[pl.ds(s * rpt, rpt)],
                        acc.at[pl.ds(s * rpt, rpt)])
        plsc.subcore_barrier()

        def make_step(half):
            base = half * H

            def step(t, carry):
                j0 = base + t * 2
                for b in range(2):
                    j = j0 + b
                    pltpu.make_async_copy(table_hbm.at[srcblk.at[j]],
                                          rbs[b], gsems[b]).wait()
                    pltpu.async_copy(rbs[b], acc.at[dstblk.at[j - base]],
                                     ssems[b], add=True)
                for b in range(2):
                    j = j0 + b
                    pltpu.make_async_copy(rbs[b], acc.at[dstblk.at[j - base]],
                                          ssems[b]).wait()
                    nxt = j + 2

                    @pl.when(nxt < NCHUNK)
                    def _():
                        pltpu.async_copy(table_hbm.at[srcblk.at[nxt]],
                                         rbs[b], gsems[b])

                return carry

            return step

        lax.fori_loop(0, H // 2, make_step(0), 0)
        pltpu.sync_copy(dst_hbm.at[w, pl.ds(H, H)], dstblk)
        lax.fori_loop(0, H // 2, make_step(1), 0)
        plsc.subcore_barrier()
        pltpu.sync_copy(acc.at[pl.ds(s * rpt, rpt)],
                        out_hbm.at[c, pl.ds(s * rpt, rpt)])

    return agg


def _pre_body(degp_ref, x_ref, w1_ref, scaled_ref, dis_ref):
    N = x_ref.shape[0]
    cnt = degp_ref[0, 0:N, 0:1] + degp_ref[1, 0:N, 0:1]
    dis = lax.rsqrt(cnt + 1.0)
    dis_ref[...] = dis
    xw = jnp.dot(x_ref[...], w1_ref[...], preferred_element_type=jnp.float32)
    scaled_ref[...] = xw * dis


def _bn_prelu(p_ref, scaled_ref, dis_ref, b_ref, g_ref, be_ref, a_ref):
    dis = dis_ref[...]
    N = scaled_ref.shape[0]
    h = (p_ref[0, 0:N] + p_ref[1, 0:N] + scaled_ref[...]) * dis + b_ref[...]
    mean = jnp.mean(h, axis=0, keepdims=True)
    d = h - mean
    var = jnp.mean(d * d, axis=0, keepdims=True)
    hn = d * lax.rsqrt(var + EPS) * g_ref[...] + be_ref[...]
    aa = a_ref[...]
    return jnp.where(hn >= 0.0, hn, aa * hn), dis


def _mid_body(p_ref, scaled_ref, dis_ref, b_ref, g_ref, be_ref, a_ref,
              w2_ref, out_ref):
    h, dis = _bn_prelu(p_ref, scaled_ref, dis_ref, b_ref, g_ref, be_ref, a_ref)
    out_ref[...] = jnp.dot(h, w2_ref[...],
                           preferred_element_type=jnp.float32) * dis


def _post_body(p_ref, scaled_ref, dis_ref, b_ref, g_ref, be_ref, a_ref,
               out_ref):
    h, _ = _bn_prelu(p_ref, scaled_ref, dis_ref, b_ref, g_ref, be_ref, a_ref)
    out_ref[...] = h


def kernel(x, edge_index, W1, b1, gamma1, beta1, a1, W2, b2, gamma2, beta2, a2):
    N, _ = x.shape
    D = W1.shape[1]
    E = edge_index.shape[1]
    info = plsc.get_sparse_core_info()
    NC, NS = info.num_cores, info.num_subcores
    NW = NC * NS
    NCHUNK = -(-E // (NW * CHUNK))
    NCHUNK = -(-NCHUNK // 4) * 4
    EPAD = NW * NCHUNK * CHUNK
    NROWS = -(-(N + 1) // (NS * 8)) * (NS * 8)

    src = edge_index[0].astype(jnp.int32)
    dst = edge_index[1].astype(jnp.int32)
    src3 = jnp.concatenate(
        [src, jnp.zeros((EPAD - E,), jnp.int32)]).reshape(NW, NCHUNK, CHUNK)
    dst3 = jnp.concatenate(
        [dst, jnp.full((EPAD - E,), N, jnp.int32)]).reshape(NW, NCHUNK, CHUNK)
    zeros128 = jnp.zeros((NROWS, D), jnp.float32)
    ones_rows = jnp.ones((CHUNK, D), jnp.float32)

    degp = _deg_kernel(NROWS, D, NC, NS, NCHUNK)(dst3, ones_rows, zeros128)

    scaled1, dis = pl.pallas_call(
        _pre_body,
        out_shape=[jax.ShapeDtypeStruct((N, D), jnp.float32),
                   jax.ShapeDtypeStruct((N, 1), jnp.float32)],
    )(degp, x, W1)

    agg = _agg_kernel(NROWS, D, NC, NS, NCHUNK)
    b1r, g1r, be1r = b1.reshape(1, D), gamma1.reshape(1, D), beta1.reshape(1, D)
    b2r, g2r, be2r = b2.reshape(1, D), gamma2.reshape(1, D), beta2.reshape(1, D)
    a1r, a2r = a1.reshape(1, 1), a2.reshape(1, 1)

    p1 = agg(scaled1, src3, dst3, zeros128)
    scaled2 = pl.pallas_call(
        _mid_body,
        out_shape=jax.ShapeDtypeStruct((N, D), jnp.float32),
    )(p1, scaled1, dis, b1r, g1r, be1r, a1r, W2)

    p2 = agg(scaled2, src3, dst3, zeros128)
    out = pl.pallas_call(
        _post_body,
        out_shape=jax.ShapeDtypeStruct((N, D), jnp.float32),
    )(p2, scaled2, dis, b2r, g2r, be2r, a2r)

    return out


# speedup vs baseline: 9.9087x; 1.0826x over previous
import functools

import jax
import jax.numpy as jnp
from jax import lax
from jax.experimental import pallas as pl
from jax.experimental.pallas import tpu as pltpu
from jax.experimental.pallas import tpu_sc as plsc

EPS = 1e-5
LANES = 16
CHUNK = 128


def _sc_mesh():
    return plsc.VectorSubcoreMesh(core_axis_name="c", subcore_axis_name="s")


@functools.lru_cache(maxsize=None)
def _deg_kernel(NROWS, D, NC, NS, NCHUNK):
    rpt = NROWS // NS

    @functools.partial(
        pl.kernel,
        mesh=_sc_mesh(),
        out_type=jax.ShapeDtypeStruct((NC, NROWS, D), jnp.float32),
        scratch_types=[
            pltpu.VMEM((NCHUNK, CHUNK), jnp.int32),
            pltpu.VMEM((CHUNK, D), jnp.float32),
            pltpu.VMEM_SHARED((NROWS, D), jnp.float32),
        ],
    )
    def deg(dst_hbm, ones_hbm, zeros_hbm, out_hbm, dstblk, ones_v, acc):
        c = lax.axis_index("c")
        s = lax.axis_index("s")
        w = c * NS + s
        pltpu.sync_copy(dst_hbm.at[w], dstblk)
        pltpu.sync_copy(ones_hbm, ones_v)
        pltpu.sync_copy(zeros_hbm.at[pl.ds(s * rpt, rpt)],
                        acc.at[pl.ds(s * rpt, rpt)])
        plsc.subcore_barrier()

        def step(j, carry):
            pltpu.sync_copy(ones_v, acc.at[dstblk.at[j]], add=True)
            return carry

        lax.fori_loop(0, NCHUNK, step, 0)
        plsc.subcore_barrier()
        pltpu.sync_copy(acc.at[pl.ds(s * rpt, rpt)],
                        out_hbm.at[c, pl.ds(s * rpt, rpt)])

    return deg


SEG = 32


@functools.lru_cache(maxsize=None)
def _agg_kernel(NROWS, D, NC, NS, NCH0, NCH1):
    rpt = NROWS // NS

    @functools.partial(
        pl.kernel,
        mesh=_sc_mesh(),
        out_type=jax.ShapeDtypeStruct((NC, NROWS, D), jnp.float32),
        scratch_types=[
            pltpu.VMEM((SEG, CHUNK), jnp.int32),
            pltpu.VMEM((SEG, CHUNK), jnp.int32),
            pltpu.VMEM((CHUNK, D), jnp.float32),
            pltpu.VMEM((CHUNK, D), jnp.float32),
            pltpu.VMEM_SHARED((NROWS, D), jnp.float32),
            pltpu.SemaphoreType.DMA,
            pltpu.SemaphoreType.DMA,
            pltpu.SemaphoreType.DMA,
            pltpu.SemaphoreType.DMA,
        ],
    )
    def agg(table_hbm, src0_hbm, dst0_hbm, src1_hbm, dst1_hbm, zeros_hbm,
            out_hbm, srcseg, dstseg, rb0, rb1, acc, gs0, gs1, ss0, ss1):
        rbs = (rb0, rb1)
        gsems = (gs0, gs1)
        ssems = (ss0, ss1)
        c = lax.axis_index("c")
        s = lax.axis_index("s")
        pltpu.sync_copy(zeros_hbm.at[pl.ds(s * rpt, rpt)],
                        acc.at[pl.ds(s * rpt, rpt)])
        plsc.subcore_barrier()

        def run(nch, srcH, dstH):
            def seg_body(g, carry):
                pltpu.sync_copy(srcH.at[s, pl.ds(g * SEG, SEG)], srcseg)
                pltpu.sync_copy(dstH.at[s, pl.ds(g * SEG, SEG)], dstseg)
                for b in range(2):
                    pltpu.async_copy(table_hbm.at[srcseg.at[b]],
                                     rbs[b], gsems[b])

                def pair(t, carry2):
                    j0 = t * 2
                    for b in range(2):
                        j = j0 + b
                        pltpu.make_async_copy(table_hbm.at[srcseg.at[j]],
                                              rbs[b], gsems[b]).wait()
                        pltpu.async_copy(rbs[b], acc.at[dstseg.at[j]],
                                         ssems[b], add=True)
                    for b in range(2):
                        j = j0 + b
                        pltpu.make_async_copy(rbs[b], acc.at[dstseg.at[j]],
                                              ssems[b]).wait()
                        nxt = j + 2

                        @pl.when(nxt < SEG)
                        def _():
                            pltpu.async_copy(table_hbm.at[srcseg.at[nxt]],
                                             rbs[b], gsems[b])

                    return carry2

                return lax.fori_loop(0, SEG // 2, pair, carry)

            lax.fori_loop(0, nch // SEG, seg_body, 0)

        @pl.when(c == 0)
        def _():
            run(NCH0, src0_hbm, dst0_hbm)

        @pl.when(c == 1)
        def _():
            run(NCH1, src1_hbm, dst1_hbm)

        plsc.subcore_barrier()
        pltpu.sync_copy(acc.at[pl.ds(s * rpt, rpt)],
                        out_hbm.at[c, pl.ds(s * rpt, rpt)])

    return agg


def _pre_body(degp_ref, x_ref, w1_ref, scaled_ref, dis_ref):
    N = x_ref.shape[0]
    cnt = degp_ref[0, 0:N, 0:1] + degp_ref[1, 0:N, 0:1]
    dis = lax.rsqrt(cnt + 1.0)
    dis_ref[...] = dis
    xw = jnp.dot(x_ref[...], w1_ref[...], preferred_element_type=jnp.float32)
    scaled_ref[...] = xw * dis


def _bn_prelu(p_ref, scaled_ref, dis_ref, b_ref, g_ref, be_ref, a_ref):
    dis = dis_ref[...]
    N = scaled_ref.shape[0]
    h = (p_ref[0, 0:N] + p_ref[1, 0:N] + scaled_ref[...]) * dis + b_ref[...]
    mean = jnp.mean(h, axis=0, keepdims=True)
    d = h - mean
    var = jnp.mean(d * d, axis=0, keepdims=True)
    hn = d * lax.rsqrt(var + EPS) * g_ref[...] + be_ref[...]
    aa = a_ref[...]
    return jnp.where(hn >= 0.0, hn, aa * hn), dis


def _mid_body(p_ref, scaled_ref, dis_ref, b_ref, g_ref, be_ref, a_ref,
              w2_ref, out_ref):
    h, dis = _bn_prelu(p_ref, scaled_ref, dis_ref, b_ref, g_ref, be_ref, a_ref)
    out_ref[...] = jnp.dot(h, w2_ref[...],
                           preferred_element_type=jnp.float32) * dis


def _post_body(p_ref, scaled_ref, dis_ref, b_ref, g_ref, be_ref, a_ref,
               out_ref):
    h, _ = _bn_prelu(p_ref, scaled_ref, dis_ref, b_ref, g_ref, be_ref, a_ref)
    out_ref[...] = h


def kernel(x, edge_index, W1, b1, gamma1, beta1, a1, W2, b2, gamma2, beta2, a2):
    N, _ = x.shape
    D = W1.shape[1]
    E = edge_index.shape[1]
    info = plsc.get_sparse_core_info()
    NC, NS = info.num_cores, info.num_subcores
    NW = NC * NS
    NCHUNK = -(-E // (NW * CHUNK))
    EPAD = NW * NCHUNK * CHUNK
    NROWS = -(-(N + 1) // (NS * 8)) * (NS * 8)

    src = edge_index[0].astype(jnp.int32)
    dst = edge_index[1].astype(jnp.int32)
    dst3 = jnp.concatenate(
        [dst, jnp.full((EPAD - E,), N, jnp.int32)]).reshape(NW, NCHUNK, CHUNK)
    NCH0 = max(SEG, int(round(0.8 * E / CHUNK / NS / SEG)) * SEG)
    n0 = min(E, NS * NCH0 * CHUNK)
    NCH1 = max(SEG, -(-(E - n0) // (NS * CHUNK * SEG)) * SEG)
    pad0 = NS * NCH0 * CHUNK - n0
    pad1 = NS * NCH1 * CHUNK - (E - n0)

    def _slabs(idx, fill, nch, lo, hi, pad):
        return jnp.concatenate(
            [idx[lo:hi], jnp.full((pad,), fill, jnp.int32)]
        ).reshape(NS, nch, CHUNK)

    src0 = _slabs(src, 0, NCH0, 0, n0, pad0)
    dst0 = _slabs(dst, N, NCH0, 0, n0, pad0)
    src1 = _slabs(src, 0, NCH1, n0, E, pad1)
    dst1 = _slabs(dst, N, NCH1, n0, E, pad1)

    zeros128 = jnp.zeros((NROWS, D), jnp.float32)
    ones_rows = jnp.ones((CHUNK, D), jnp.float32)

    degp = _deg_kernel(NROWS, D, NC, NS, NCHUNK)(dst3, ones_rows, zeros128)

    scaled1, dis = pl.pallas_call(
        _pre_body,
        out_shape=[jax.ShapeDtypeStruct((N, D), jnp.float32),
                   jax.ShapeDtypeStruct((N, 1), jnp.float32)],
    )(degp, x, W1)

    agg = _agg_kernel(NROWS, D, NC, NS, NCH0, NCH1)
    b1r, g1r, be1r = b1.reshape(1, D), gamma1.reshape(1, D), beta1.reshape(1, D)
    b2r, g2r, be2r = b2.reshape(1, D), gamma2.reshape(1, D), beta2.reshape(1, D)
    a1r, a2r = a1.reshape(1, 1), a2.reshape(1, 1)

    p1 = agg(scaled1, src0, dst0, src1, dst1, zeros128)
    scaled2 = pl.pallas_call(
        _mid_body,
        out_shape=jax.ShapeDtypeStruct((N, D), jnp.float32),
    )(p1, scaled1, dis, b1r, g1r, be1r, a1r, W2)

    p2 = agg(scaled2, src0, dst0, src1, dst1, zeros128)
    out = pl.pallas_call(
        _post_body,
        out_shape=jax.ShapeDtypeStruct((N, D), jnp.float32),
    )(p2, scaled2, dis, b2r, g2r, be2r, a2r)

    return out


# speedup vs baseline: 26.6081x; 2.6853x over previous
import functools

import jax
import jax.numpy as jnp
from jax import lax
from jax.experimental import pallas as pl
from jax.experimental.pallas import tpu as pltpu
from jax.experimental.pallas import tpu_sc as plsc

EPS = 1e-5
LANES = 16
CHUNK = 128


def _sc_mesh():
    return plsc.VectorSubcoreMesh(core_axis_name="c", subcore_axis_name="s")


@functools.lru_cache(maxsize=None)
def _deg_kernel(NROWS, D, NC, NS, NCHUNK):
    rpt = NROWS // NS

    @functools.partial(
        pl.kernel,
        mesh=_sc_mesh(),
        out_type=jax.ShapeDtypeStruct((NC, NROWS, D), jnp.float32),
        scratch_types=[
            pltpu.VMEM((NCHUNK, CHUNK), jnp.int32),
            pltpu.VMEM((CHUNK, D), jnp.float32),
            pltpu.VMEM_SHARED((NROWS, D), jnp.float32),
        ],
    )
    def deg(dst_hbm, ones_hbm, zeros_hbm, out_hbm, dstblk, ones_v, acc):
        c = lax.axis_index("c")
        s = lax.axis_index("s")
        w = c * NS + s
        pltpu.sync_copy(dst_hbm.at[w], dstblk)
        pltpu.sync_copy(ones_hbm, ones_v)
        pltpu.sync_copy(zeros_hbm.at[pl.ds(s * rpt, rpt)],
                        acc.at[pl.ds(s * rpt, rpt)])
        plsc.subcore_barrier()

        def step(j, carry):
            pltpu.sync_copy(ones_v, acc.at[dstblk.at[j]], add=True)
            return carry

        lax.fori_loop(0, NCHUNK, step, 0)
        plsc.subcore_barrier()
        pltpu.sync_copy(acc.at[pl.ds(s * rpt, rpt)],
                        out_hbm.at[c, pl.ds(s * rpt, rpt)])

    return deg


SEG = 16


@functools.lru_cache(maxsize=None)
def _agg_kernel(NROWS, D, NC, NS, NCHUNK):
    rpt = NROWS // NS

    @functools.partial(
        pl.kernel,
        mesh=_sc_mesh(),
        out_type=jax.ShapeDtypeStruct((NC, NROWS, D), jnp.float32),
        scratch_types=[
            pltpu.VMEM((SEG, CHUNK), jnp.int32),
            pltpu.VMEM((SEG, CHUNK), jnp.int32),
            pltpu.VMEM((CHUNK, D), jnp.float32),
            pltpu.VMEM((CHUNK, D), jnp.float32),
            pltpu.VMEM_SHARED((NROWS, D), jnp.float32),
            pltpu.SemaphoreType.DMA,
            pltpu.SemaphoreType.DMA,
        ],
    )
    def agg(table_hbm, src_hbm, dst_hbm, zeros_hbm,
            out_hbm, srcseg, dstseg, rb0, rb1, acc, gs0, gs1):
        rbs = (rb0, rb1)
        gsems = (gs0, gs1)
        c = lax.axis_index("c")
        s = lax.axis_index("s")
        w = c * NS + s
        pltpu.sync_copy(zeros_hbm.at[pl.ds(s * rpt, rpt)],
                        acc.at[pl.ds(s * rpt, rpt)])
        plsc.subcore_barrier()

        def seg_body(g, carry):
            pltpu.sync_copy(src_hbm.at[w, pl.ds(g * SEG, SEG)], srcseg)
            pltpu.sync_copy(dst_hbm.at[w, pl.ds(g * SEG, SEG)], dstseg)
            for b in range(2):
                pltpu.async_copy(table_hbm.at[srcseg.at[b]],
                                 rbs[b], gsems[b])

            def pair(t, carry2):
                j0 = t * 2
                for b in range(2):
                    j = j0 + b
                    pltpu.make_async_copy(table_hbm.at[srcseg.at[j]],
                                          rbs[b], gsems[b]).wait()
                    pltpu.sync_copy(rbs[b], acc.at[dstseg.at[j]], add=True)
                    nxt = j + 2

                    @pl.when(nxt < SEG)
                    def _():
                        pltpu.async_copy(table_hbm.at[srcseg.at[nxt]],
                                         rbs[b], gsems[b])

                return carry2

            return lax.fori_loop(0, SEG // 2, pair, carry)

        lax.fori_loop(0, NCHUNK // SEG, seg_body, 0)
        plsc.subcore_barrier()
        pltpu.sync_copy(acc.at[pl.ds(s * rpt, rpt)],
                        out_hbm.at[c, pl.ds(s * rpt, rpt)])

    return agg


def _pre_body(degp_ref, x_ref, w1_ref, scaled_ref, dis_ref):
    N = x_ref.shape[0]
    cnt = degp_ref[0, 0:N, 0:1] + degp_ref[1, 0:N, 0:1]
    dis = lax.rsqrt(cnt + 1.0)
    dis_ref[...] = dis
    xw = jnp.dot(x_ref[...], w1_ref[...], preferred_element_type=jnp.float32)
    scaled_ref[...] = xw * dis


def _bn_prelu(p_ref, scaled_ref, dis_ref, b_ref, g_ref, be_ref, a_ref):
    dis = dis_ref[...]
    N = scaled_ref.shape[0]
    h = (p_ref[0, 0:N] + p_ref[1, 0:N] + scaled_ref[...]) * dis + b_ref[...]
    mean = jnp.mean(h, axis=0, keepdims=True)
    d = h - mean
    var = jnp.mean(d * d, axis=0, keepdims=True)
    hn = d * lax.rsqrt(var + EPS) * g_ref[...] + be_ref[...]
    aa = a_ref[...]
    return jnp.where(hn >= 0.0, hn, aa * hn), dis


def _mid_body(p_ref, scaled_ref, dis_ref, b_ref, g_ref, be_ref, a_ref,
              w2_ref, out_ref):
    h, dis = _bn_prelu(p_ref, scaled_ref, dis_ref, b_ref, g_ref, be_ref, a_ref)
    out_ref[...] = jnp.dot(h, w2_ref[...],
                           preferred_element_type=jnp.float32) * dis


def _post_body(p_ref, scaled_ref, dis_ref, b_ref, g_ref, be_ref, a_ref,
               out_ref):
    h, _ = _bn_prelu(p_ref, scaled_ref, dis_ref, b_ref, g_ref, be_ref, a_ref)
    out_ref[...] = h


def kernel(x, edge_index, W1, b1, gamma1, beta1, a1, W2, b2, gamma2, beta2, a2):
    N, _ = x.shape
    D = W1.shape[1]
    E = edge_index.shape[1]
    info = plsc.get_sparse_core_info()
    NC, NS = info.num_cores, info.num_subcores
    NW = NC * NS
    NCHUNK = -(-E // (NW * CHUNK))
    NCHUNK = -(-NCHUNK // SEG) * SEG
    EPAD = NW * NCHUNK * CHUNK
    NROWS = -(-(N + 1) // (NS * 8)) * (NS * 8)

    src = edge_index[0].astype(jnp.int32)
    dst = edge_index[1].astype(jnp.int32)
    npad = EPAD - E
    pad_src = (jnp.arange(npad, dtype=jnp.int32) * 37) % N
    pad_dst = N + (jnp.arange(npad, dtype=jnp.int32) % (NROWS - N))
    src3 = jnp.concatenate([src, pad_src]).reshape(NW, NCHUNK, CHUNK)
    dst3 = jnp.concatenate([dst, pad_dst]).reshape(NW, NCHUNK, CHUNK)

    zeros128 = jnp.zeros((NROWS, D), jnp.float32)
    ones_rows = jnp.ones((CHUNK, D), jnp.float32)

    degp = _deg_kernel(NROWS, D, NC, NS, NCHUNK)(dst3, ones_rows, zeros128)

    scaled1, dis = pl.pallas_call(
        _pre_body,
        out_shape=[jax.ShapeDtypeStruct((N, D), jnp.float32),
                   jax.ShapeDtypeStruct((N, 1), jnp.float32)],
    )(degp, x, W1)

    agg = _agg_kernel(NROWS, D, NC, NS, NCHUNK)
    b1r, g1r, be1r = b1.reshape(1, D), gamma1.reshape(1, D), beta1.reshape(1, D)
    b2r, g2r, be2r = b2.reshape(1, D), gamma2.reshape(1, D), beta2.reshape(1, D)
    a1r, a2r = a1.reshape(1, 1), a2.reshape(1, 1)

    p1 = agg(scaled1, src3, dst3, zeros128)
    scaled2 = pl.pallas_call(
        _mid_body,
        out_shape=jax.ShapeDtypeStruct((N, D), jnp.float32),
    )(p1, scaled1, dis, b1r, g1r, be1r, a1r, W2)

    p2 = agg(scaled2, src3, dst3, zeros128)
    out = pl.pallas_call(
        _post_body,
        out_shape=jax.ShapeDtypeStruct((N, D), jnp.float32),
    )(p2, scaled2, dis, b2r, g2r, be2r, a2r)

    return out


# speedup vs baseline: 27.8762x; 1.0477x over previous
import functools

import jax
import jax.numpy as jnp
from jax import lax
from jax.experimental import pallas as pl
from jax.experimental.pallas import tpu as pltpu
from jax.experimental.pallas import tpu_sc as plsc

EPS = 1e-5
LANES = 16
CHUNK = 128


def _sc_mesh():
    return plsc.VectorSubcoreMesh(core_axis_name="c", subcore_axis_name="s")


@functools.lru_cache(maxsize=None)
def _deg_kernel(NROWS, D, NC, NS, NCHUNK):
    rpt = NROWS // NS

    @functools.partial(
        pl.kernel,
        mesh=_sc_mesh(),
        out_type=jax.ShapeDtypeStruct((NC, NROWS, D), jnp.float32),
        scratch_types=[
            pltpu.VMEM((NCHUNK, CHUNK), jnp.int32),
            pltpu.VMEM((CHUNK, D), jnp.float32),
            pltpu.VMEM_SHARED((NROWS, D), jnp.float32),
        ],
    )
    def deg(dst_hbm, ones_hbm, zeros_hbm, out_hbm, dstblk, ones_v, acc):
        c = lax.axis_index("c")
        s = lax.axis_index("s")
        w = c * NS + s
        pltpu.sync_copy(dst_hbm.at[w], dstblk)
        pltpu.sync_copy(ones_hbm, ones_v)
        pltpu.sync_copy(zeros_hbm.at[pl.ds(s * rpt, rpt)],
                        acc.at[pl.ds(s * rpt, rpt)])
        plsc.subcore_barrier()

        def step(j, carry):
            pltpu.sync_copy(ones_v, acc.at[dstblk.at[j]], add=True)
            return carry

        lax.fori_loop(0, NCHUNK, step, 0)
        plsc.subcore_barrier()
        pltpu.sync_copy(acc.at[pl.ds(s * rpt, rpt)],
                        out_hbm.at[c, pl.ds(s * rpt, rpt)])

    return deg


SEG = 40


@functools.lru_cache(maxsize=None)
def _agg_kernel(NROWS, D, NC, NS, NCHUNK):
    rpt = NROWS // NS

    @functools.partial(
        pl.kernel,
        mesh=_sc_mesh(),
        out_type=jax.ShapeDtypeStruct((NC, NROWS, D), jnp.float32),
        scratch_types=[
            pltpu.VMEM((SEG, CHUNK), jnp.int32),
            pltpu.VMEM((SEG, CHUNK), jnp.int32),
            pltpu.VMEM((CHUNK, D), jnp.float32),
            pltpu.VMEM((CHUNK, D), jnp.float32),
            pltpu.VMEM_SHARED((NROWS, D), jnp.float32),
            pltpu.SemaphoreType.DMA,
            pltpu.SemaphoreType.DMA,
        ],
    )
    def agg(table_hbm, src_hbm, dst_hbm, zeros_hbm,
            out_hbm, srcseg, dstseg, rb0, rb1, acc, gs0, gs1):
        rbs = (rb0, rb1)
        gsems = (gs0, gs1)
        c = lax.axis_index("c")
        s = lax.axis_index("s")
        w = c * NS + s
        pltpu.sync_copy(zeros_hbm.at[pl.ds(s * rpt, rpt)],
                        acc.at[pl.ds(s * rpt, rpt)])
        plsc.subcore_barrier()

        def seg_body(g, carry):
            pltpu.sync_copy(src_hbm.at[w, pl.ds(g * SEG, SEG)], srcseg)
            pltpu.sync_copy(dst_hbm.at[w, pl.ds(g * SEG, SEG)], dstseg)
            for b in range(2):
                pltpu.async_copy(table_hbm.at[srcseg.at[b]],
                                 rbs[b], gsems[b])

            def pair(t, carry2):
                j0 = t * 2
                for b in range(2):
                    j = j0 + b
                    pltpu.make_async_copy(table_hbm.at[srcseg.at[j]],
                                          rbs[b], gsems[b]).wait()
                    pltpu.sync_copy(rbs[b], acc.at[dstseg.at[j]], add=True)
                    nxt = j + 2

                    @pl.when(nxt < SEG)
                    def _():
                        pltpu.async_copy(table_hbm.at[srcseg.at[nxt]],
                                         rbs[b], gsems[b])

                return carry2

            return lax.fori_loop(0, SEG // 2, pair, carry)

        lax.fori_loop(0, NCHUNK // SEG, seg_body, 0)
        plsc.subcore_barrier()
        pltpu.sync_copy(acc.at[pl.ds(s * rpt, rpt)],
                        out_hbm.at[c, pl.ds(s * rpt, rpt)])

    return agg


def _pre_body(degp_ref, x_ref, w1_ref, scaled_ref, dis_ref):
    N = x_ref.shape[0]
    cnt = degp_ref[0, 0:N, 0:1] + degp_ref[1, 0:N, 0:1]
    dis = lax.rsqrt(cnt + 1.0)
    dis_ref[...] = dis
    xw = jnp.dot(x_ref[...], w1_ref[...], preferred_element_type=jnp.float32)
    scaled_ref[...] = xw * dis


def _bn_prelu(p_ref, scaled_ref, dis_ref, b_ref, g_ref, be_ref, a_ref):
    dis = dis_ref[...]
    N = scaled_ref.shape[0]
    h = (p_ref[0, 0:N] + p_ref[1, 0:N] + scaled_ref[...]) * dis + b_ref[...]
    mean = jnp.mean(h, axis=0, keepdims=True)
    d = h - mean
    var = jnp.mean(d * d, axis=0, keepdims=True)
    hn = d * lax.rsqrt(var + EPS) * g_ref[...] + be_ref[...]
    aa = a_ref[...]
    return jnp.where(hn >= 0.0, hn, aa * hn), dis


def _mid_body(p_ref, scaled_ref, dis_ref, b_ref, g_ref, be_ref, a_ref,
              w2_ref, out_ref):
    h, dis = _bn_prelu(p_ref, scaled_ref, dis_ref, b_ref, g_ref, be_ref, a_ref)
    out_ref[...] = jnp.dot(h, w2_ref[...],
                           preferred_element_type=jnp.float32) * dis


def _post_body(p_ref, scaled_ref, dis_ref, b_ref, g_ref, be_ref, a_ref,
               out_ref):
    h, _ = _bn_prelu(p_ref, scaled_ref, dis_ref, b_ref, g_ref, be_ref, a_ref)
    out_ref[...] = h


def kernel(x, edge_index, W1, b1, gamma1, beta1, a1, W2, b2, gamma2, beta2, a2):
    N, _ = x.shape
    D = W1.shape[1]
    E = edge_index.shape[1]
    info = plsc.get_sparse_core_info()
    NC, NS = info.num_cores, info.num_subcores
    NW = NC * NS
    NCHUNK = -(-E // (NW * CHUNK))
    NCHUNK = -(-NCHUNK // SEG) * SEG
    EPAD = NW * NCHUNK * CHUNK
    NROWS = -(-(N + 1) // (NS * 8)) * (NS * 8)

    src = edge_index[0].astype(jnp.int32)
    dst = edge_index[1].astype(jnp.int32)
    npad = EPAD - E
    pad_src = (jnp.arange(npad, dtype=jnp.int32) * 37) % N
    pad_dst = N + (jnp.arange(npad, dtype=jnp.int32) % (NROWS - N))
    src3 = jnp.concatenate([src, pad_src]).reshape(NW, NCHUNK, CHUNK)
    dst3 = jnp.concatenate([dst, pad_dst]).reshape(NW, NCHUNK, CHUNK)

    zeros128 = jnp.zeros((NROWS, D), jnp.float32)
    ones_rows = jnp.ones((CHUNK, D), jnp.float32)

    degp = _deg_kernel(NROWS, D, NC, NS, NCHUNK)(dst3, ones_rows, zeros128)

    scaled1, dis = pl.pallas_call(
        _pre_body,
        out_shape=[jax.ShapeDtypeStruct((N, D), jnp.float32),
                   jax.ShapeDtypeStruct((N, 1), jnp.float32)],
    )(degp, x, W1)

    agg = _agg_kernel(NROWS, D, NC, NS, NCHUNK)
    b1r, g1r, be1r = b1.reshape(1, D), gamma1.reshape(1, D), beta1.reshape(1, D)
    b2r, g2r, be2r = b2.reshape(1, D), gamma2.reshape(1, D), beta2.reshape(1, D)
    a1r, a2r = a1.reshape(1, 1), a2.reshape(1, 1)

    p1 = agg(scaled1, src3, dst3, zeros128)
    scaled2 = pl.pallas_call(
        _mid_body,
        out_shape=jax.ShapeDtypeStruct((N, D), jnp.float32),
    )(p1, scaled1, dis, b1r, g1r, be1r, a1r, W2)

    p2 = agg(scaled2, src3, dst3, zeros128)
    out = pl.pallas_call(
        _post_body,
        out_shape=jax.ShapeDtypeStruct((N, D), jnp.float32),
    )(p2, scaled2, dis, b2r, g2r, be2r, a2r)

    return out
